# baseline (speedup 1.0000x reference)
import functools

import jax
import jax.numpy as jnp
from jax import lax
from jax.experimental import pallas as pl
from jax.experimental.pallas import tpu as pltpu
from jax.experimental.pallas import tpu_sc as plsc

N_NODES = 10000
N_EDGES = 320000
FEATS = 128

NC = 2
NS = 16
NW = NC * NS
CHUNK = 128
NCHUNK = 2 * (-(-N_EDGES // (NW * CHUNK * 2)))
NHALF = NCHUNK // 2
E_PAD = NW * NCHUNK * CHUNK
NPT = (-(-N_NODES // NS) + 7) // 8 * 8
N_PAD = NPT * NS


def _scatter_body(src_hbm, dst_hbm, x_hbm, zeros_hbm, out_hbm,
                  srcall_v, dst0_v, dst1_v, rows0_v, rows1_v, acc_s,
                  sem_g0, sem_g1, sem_i0, sem_i1):
    cid = lax.axis_index("c")
    sid = lax.axis_index("s")
    wid = cid * NS + sid

    pltpu.sync_copy(zeros_hbm, acc_s.at[pl.ds(sid * NPT, NPT)])
    pltpu.sync_copy(src_hbm.at[wid], srcall_v)
    plsc.subcore_barrier()

    def gather(c, rows_v, sem):
        return pltpu.async_copy(x_hbm.at[srcall_v.at[c]], rows_v, sem)

    def gather_wait(c, rows_v, sem):
        pltpu.make_async_copy(x_hbm.at[srcall_v.at[c]], rows_v, sem).wait()

    def idx(c, dst_v, sem):
        return pltpu.async_copy(dst_hbm.at[wid, c], dst_v, sem)

    def idx_wait(c, dst_v, sem):
        pltpu.make_async_copy(dst_hbm.at[wid, c], dst_v, sem).wait()

    gather(0, rows0_v, sem_g0)
    idx(0, dst0_v, sem_i0)

    def body(k, carry):
        c = 2 * k
        gather(c + 1, rows1_v, sem_g1)
        idx(c + 1, dst1_v, sem_i1)
        gather_wait(c, rows0_v, sem_g0)
        idx_wait(c, dst0_v, sem_i0)
        pltpu.sync_copy(rows0_v, acc_s.at[dst0_v], add=True)

        @pl.when(k < NHALF - 1)
        def _():
            gather(c + 2, rows0_v, sem_g0)
            idx(c + 2, dst0_v, sem_i0)

        gather_wait(c + 1, rows1_v, sem_g1)
        idx_wait(c + 1, dst1_v, sem_i1)
        pltpu.sync_copy(rows1_v, acc_s.at[dst1_v], add=True)
        return carry

    lax.fori_loop(0, NHALF, body, 0)
    plsc.subcore_barrier()

    pltpu.sync_copy(acc_s.at[pl.ds(sid * NPT, NPT)],
                    out_hbm.at[cid, pl.ds(sid * NPT, NPT)])


_scatter_sc = functools.partial(
    pl.kernel,
    mesh=plsc.VectorSubcoreMesh(core_axis_name="c", subcore_axis_name="s"),
    out_type=jax.ShapeDtypeStruct((NC, N_PAD, FEATS), jnp.float32),
    scratch_types=[
        pltpu.VMEM((NCHUNK, CHUNK), jnp.int32),
        pltpu.VMEM((CHUNK,), jnp.int32),
        pltpu.VMEM((CHUNK,), jnp.int32),
        pltpu.VMEM((CHUNK, FEATS), jnp.float32),
        pltpu.VMEM((CHUNK, FEATS), jnp.float32),
        pltpu.VMEM_SHARED((N_PAD, FEATS), jnp.float32),
        pltpu.SemaphoreType.DMA,
        pltpu.SemaphoreType.DMA,
        pltpu.SemaphoreType.DMA,
        pltpu.SemaphoreType.DMA,
    ],
)(_scatter_body)


def _linear_body(p0_ref, p1_ref, wt_ref, b_ref, o_ref):
    h = p0_ref[...] + p1_ref[...]
    o_ref[...] = (
        jnp.dot(h, wt_ref[...], preferred_element_type=jnp.float32) + b_ref[...]
    )


def _linear_tc(p0, p1, wt, b2):
    m = p0.shape[0]
    bm = 1000
    return pl.pallas_call(
        _linear_body,
        grid=(m // bm,),
        in_specs=[
            pl.BlockSpec((bm, FEATS), lambda i: (i, 0)),
            pl.BlockSpec((bm, FEATS), lambda i: (i, 0)),
            pl.BlockSpec((FEATS, FEATS), lambda i: (0, 0)),
            pl.BlockSpec((1, FEATS), lambda i: (0, 0)),
        ],
        out_specs=pl.BlockSpec((bm, FEATS), lambda i: (i, 0)),
        out_shape=jax.ShapeDtypeStruct((m, FEATS), jnp.float32),
    )(p0, p1, wt, b2)


def kernel(x, edge_index, W, b):
    src = edge_index[0].astype(jnp.int32)
    dst = edge_index[1].astype(jnp.int32)
    pad = E_PAD - N_EDGES
    src = jnp.concatenate([src, jnp.zeros((pad,), jnp.int32)])
    dst = jnp.concatenate([dst, jnp.full((pad,), N_NODES, jnp.int32)])
    src3 = src.reshape(NW, NCHUNK, CHUNK)
    dst3 = dst.reshape(NW, NCHUNK, CHUNK)
    zeros = jnp.zeros((NPT, FEATS), jnp.float32)
    partial = _scatter_sc(src3, dst3, x, zeros)
    return _linear_tc(partial[0, :N_NODES], partial[1, :N_NODES],
                      W.T, b.reshape(1, FEATS))
